# Initial kernel scaffold; baseline (speedup 1.0000x reference)
import jax
import jax.numpy as jnp
from jax.experimental import pallas as pl


def kernel(node_feats, node_xy, node_adj_ids, edge_ids, W_node, b_node, W_edge, b_edge):
    raise NotImplementedError("write your pallas kernel here")


# speedup vs baseline: 1.1639x; 1.1639x over previous
import functools

import jax
import jax.numpy as jnp
from jax import lax
from jax.experimental import pallas as pl
from jax.experimental.pallas import tpu as pltpu
from jax.experimental.pallas import tpu_sc as plsc

N_NODES = 10000
N_EDGES = 320000
D_FEAT = 128

NC = 2
NS = 16
NW = NC * NS
EPW = N_EDGES // NW
BLK = 80
NBLK = EPW // BLK
GRP = BLK // 16


def _dense_body(x_ref, wn_ref, bn_ref, we_ref, be_ref, feats_ref, ns_ref):
    x = x_ref[...]
    feats = jnp.dot(x, we_ref[...], preferred_element_type=jnp.float32)
    feats_ref[...] = feats + be_ref[...]
    ns = jnp.dot(x, wn_ref[...], preferred_element_type=jnp.float32)
    ns_ref[...] = jax.nn.sigmoid(ns + bn_ref[...])


def _dense(node_feats, W_node, b_node, W_edge, b_edge):
    return pl.pallas_call(
        _dense_body,
        out_shape=(
            jax.ShapeDtypeStruct((N_NODES, D_FEAT), jnp.float32),
            jax.ShapeDtypeStruct((N_NODES, 1), jnp.float32),
        ),
    )(node_feats, W_node, b_node, W_edge, b_edge)


_mesh = plsc.VectorSubcoreMesh(core_axis_name="c", subcore_axis_name="s")


@functools.partial(
    pl.kernel,
    mesh=_mesh,
    out_type=jax.ShapeDtypeStruct((N_EDGES,), jnp.float32),
    compiler_params=pltpu.CompilerParams(needs_layout_passes=False),
    scratch_types=[
        pltpu.VMEM((EPW,), jnp.int32),
        pltpu.VMEM((EPW,), jnp.int32),
        pltpu.VMEM((BLK, D_FEAT), jnp.float32),
        pltpu.VMEM((BLK, D_FEAT), jnp.float32),
        pltpu.VMEM((EPW,), jnp.float32),
        pltpu.SemaphoreType.DMA,
    ],
)
def _edge_kernel(feats, src, dst, out, src_v, dst_v, rs_v, rd_v, out_v, sem):
    wid = lax.axis_index("s") * NC + lax.axis_index("c")
    base = wid * EPW
    pltpu.sync_copy(src.at[pl.ds(base, EPW)], src_v)
    pltpu.sync_copy(dst.at[pl.ds(base, EPW)], dst_v)

    lane = lax.iota(jnp.int32, 16)

    def block(b, carry):
        e0 = b * BLK
        cs = pltpu.async_copy(feats.at[src_v.at[pl.ds(e0, BLK)]], rs_v, sem)
        cd = pltpu.async_copy(feats.at[dst_v.at[pl.ds(e0, BLK)]], rd_v, sem)
        cs.wait()
        cd.wait()
        for g in range(GRP):
            e_vec = g * 16 + lane
            acc = jnp.zeros((16,), jnp.float32)

            def dbody(d8, acc):
                for dd in range(8):
                    d_vec = jnp.full((16,), d8 * 8 + dd, jnp.int32)
                    a = plsc.load_gather(rs_v, [e_vec, d_vec])
                    bb = plsc.load_gather(rd_v, [e_vec, d_vec])
                    acc = acc + a * bb
                return acc

            acc = lax.fori_loop(0, D_FEAT // 8, dbody, acc)
            out_v[pl.ds(e0 + g * 16, 16)] = 1.0 / (1.0 + jnp.exp(-acc))
        return carry

    lax.fori_loop(0, NBLK, block, 0)
    pltpu.sync_copy(out_v, out.at[pl.ds(base, EPW)])


def kernel(node_feats, node_xy, node_adj_ids, edge_ids, W_node, b_node,
           W_edge, b_edge):
    feats, ns = _dense(node_feats, W_node, b_node.reshape(1, 1),
                       W_edge, b_edge.reshape(1, D_FEAT))
    node_scores = ns[:, 0]
    src = edge_ids[0].astype(jnp.int32)
    dst = edge_ids[1].astype(jnp.int32)
    edge_scores = _edge_kernel(feats, src, dst)
    return (node_scores, edge_scores)


# speedup vs baseline: 1.4448x; 1.2413x over previous
import functools

import jax
import jax.numpy as jnp
from jax import lax
from jax.experimental import pallas as pl
from jax.experimental.pallas import tpu as pltpu
from jax.experimental.pallas import tpu_sc as plsc

N_NODES = 10000
N_EDGES = 320000
D_FEAT = 128

NC = 2
NS = 16
NW = NC * NS
EPW = N_EDGES // NW
BLK = 80
NBLK = EPW // BLK
GRP = BLK // 16


def _dense_body(x_ref, wn_ref, bn_ref, we_ref, be_ref, feats_ref, ns_ref):
    x = x_ref[...]
    feats = jnp.dot(x, we_ref[...], preferred_element_type=jnp.float32)
    feats_ref[...] = feats + be_ref[...]
    ns = jnp.dot(x, wn_ref[...], preferred_element_type=jnp.float32)
    ns_ref[...] = jax.nn.sigmoid(ns + bn_ref[...])


def _dense(node_feats, W_node, b_node, W_edge, b_edge):
    return pl.pallas_call(
        _dense_body,
        out_shape=(
            jax.ShapeDtypeStruct((N_NODES, D_FEAT), jnp.float32),
            jax.ShapeDtypeStruct((N_NODES, 1), jnp.float32),
        ),
    )(node_feats, W_node, b_node, W_edge, b_edge)


_mesh = plsc.VectorSubcoreMesh(core_axis_name="c", subcore_axis_name="s")


@functools.partial(
    pl.kernel,
    mesh=_mesh,
    out_type=jax.ShapeDtypeStruct((N_EDGES,), jnp.float32),
    compiler_params=pltpu.CompilerParams(needs_layout_passes=False),
    scratch_types=[
        pltpu.VMEM((EPW,), jnp.int32),
        pltpu.VMEM((EPW,), jnp.int32),
        pltpu.VMEM((BLK, D_FEAT), jnp.float32),
        pltpu.VMEM((BLK, D_FEAT), jnp.float32),
        pltpu.VMEM((BLK, D_FEAT), jnp.float32),
        pltpu.VMEM((BLK, D_FEAT), jnp.float32),
        pltpu.VMEM((EPW,), jnp.float32),
        pltpu.SemaphoreType.DMA,
        pltpu.SemaphoreType.DMA,
    ],
)
def _edge_kernel(feats, src, dst, out, src_v, dst_v,
                 rs_a, rd_a, rs_b, rd_b, out_v, sem_a, sem_b):
    wid = lax.axis_index("s") * NC + lax.axis_index("c")
    base = wid * EPW
    pltpu.sync_copy(src.at[pl.ds(base, EPW)], src_v)
    pltpu.sync_copy(dst.at[pl.ds(base, EPW)], dst_v)

    lane = lax.iota(jnp.int32, 16)

    def start(b, rs, rd, sem):
        e0 = b * BLK
        pltpu.async_copy(feats.at[src_v.at[pl.ds(e0, BLK)]], rs, sem)
        pltpu.async_copy(feats.at[dst_v.at[pl.ds(e0, BLK)]], rd, sem)

    def drain(rs, rd, sem):
        pltpu.make_async_copy(feats.at[src_v.at[pl.ds(0, BLK)]], rs, sem).wait()
        pltpu.make_async_copy(feats.at[dst_v.at[pl.ds(0, BLK)]], rd, sem).wait()

    def compute(b, rs, rd):
        e0 = b * BLK
        for g in range(GRP):
            e_vec = g * 16 + lane
            a0 = jnp.zeros((16,), jnp.float32)
            a1 = jnp.zeros((16,), jnp.float32)
            a2 = jnp.zeros((16,), jnp.float32)
            a3 = jnp.zeros((16,), jnp.float32)

            def dbody(d16, accs):
                a0, a1, a2, a3 = accs
                acc = [a0, a1, a2, a3]
                for dd in range(16):
                    d_vec = jnp.full((16,), d16 * 16 + dd, jnp.int32)
                    a = plsc.load_gather(rs, [e_vec, d_vec])
                    bb = plsc.load_gather(rd, [e_vec, d_vec])
                    acc[dd % 4] = acc[dd % 4] + a * bb
                return tuple(acc)

            a0, a1, a2, a3 = lax.fori_loop(
                0, D_FEAT // 16, dbody, (a0, a1, a2, a3))
            acc = (a0 + a1) + (a2 + a3)
            out_v[pl.ds(e0 + g * 16, 16)] = 1.0 / (1.0 + jnp.exp(-acc))

    start(0, rs_a, rd_a, sem_a)

    def pair(i, carry):
        b_even = 2 * i
        start(b_even + 1, rs_b, rd_b, sem_b)
        drain(rs_a, rd_a, sem_a)
        compute(b_even, rs_a, rd_a)
        start(b_even + 2, rs_a, rd_a, sem_a)
        drain(rs_b, rd_b, sem_b)
        compute(b_even + 1, rs_b, rd_b)
        return carry

    lax.fori_loop(0, (NBLK - 1) // 2, pair, 0)
    drain(rs_a, rd_a, sem_a)
    compute(NBLK - 1, rs_a, rd_a)
    pltpu.sync_copy(out_v, out.at[pl.ds(base, EPW)])


def kernel(node_feats, node_xy, node_adj_ids, edge_ids, W_node, b_node,
           W_edge, b_edge):
    feats, ns = _dense(node_feats, W_node, b_node.reshape(1, 1),
                       W_edge, b_edge.reshape(1, D_FEAT))
    node_scores = ns[:, 0]
    src = edge_ids[0].astype(jnp.int32)
    dst = edge_ids[1].astype(jnp.int32)
    edge_scores = _edge_kernel(feats, src, dst)
    return (node_scores, edge_scores)


# speedup vs baseline: 7.2779x; 5.0371x over previous
import functools

import jax
import jax.numpy as jnp
from jax import lax
from jax.experimental import pallas as pl
from jax.experimental.pallas import tpu as pltpu
from jax.experimental.pallas import tpu_sc as plsc

N_NODES = 10000
N_EDGES = 320000
D_FEAT = 128

NC = 2
NS = 16
NW = NC * NS
EPW = N_EDGES // NW
BLK = 80
NBLK = EPW // BLK
GRP = BLK // 16


def _dense_body(x_ref, wn_ref, bn_ref, we_ref, be_ref, feats_ref, ns_ref):
    x = x_ref[...]
    feats = jnp.dot(x, we_ref[...], preferred_element_type=jnp.float32)
    feats_ref[...] = feats + be_ref[...]
    ns = jnp.dot(x, wn_ref[...], preferred_element_type=jnp.float32)
    ns_ref[...] = jax.nn.sigmoid(ns + bn_ref[...])


def _dense(node_feats, W_node, b_node, W_edge, b_edge):
    return pl.pallas_call(
        _dense_body,
        out_shape=(
            jax.ShapeDtypeStruct((N_NODES, D_FEAT), jnp.float32),
            jax.ShapeDtypeStruct((N_NODES, 1), jnp.float32),
        ),
    )(node_feats, W_node, b_node, W_edge, b_edge)


_mesh = plsc.VectorSubcoreMesh(core_axis_name="c", subcore_axis_name="s")


@functools.partial(
    pl.kernel,
    mesh=_mesh,
    out_type=jax.ShapeDtypeStruct((N_EDGES,), jnp.float32),
    compiler_params=pltpu.CompilerParams(needs_layout_passes=False),
    scratch_types=[
        pltpu.VMEM((EPW,), jnp.int32),
        pltpu.VMEM((EPW,), jnp.int32),
        pltpu.VMEM((BLK, D_FEAT), jnp.float32),
        pltpu.VMEM((BLK, D_FEAT), jnp.float32),
        pltpu.VMEM((BLK, D_FEAT), jnp.float32),
        pltpu.VMEM((BLK, D_FEAT), jnp.float32),
        pltpu.VMEM((EPW,), jnp.float32),
        pltpu.VMEM((16 * 17,), jnp.float32),
        pltpu.SemaphoreType.DMA,
        pltpu.SemaphoreType.DMA,
    ],
)
def _edge_kernel(feats, src, dst, out, src_v, dst_v,
                 rs_a, rd_a, rs_b, rd_b, out_v, tr_v, sem_a, sem_b):
    wid = lax.axis_index("s") * NC + lax.axis_index("c")
    base = wid * EPW
    pltpu.sync_copy(src.at[pl.ds(base, EPW)], src_v)
    pltpu.sync_copy(dst.at[pl.ds(base, EPW)], dst_v)

    lane = lax.iota(jnp.int32, 16)

    def start(b, rs, rd, sem):
        e0 = b * BLK
        pltpu.async_copy(feats.at[src_v.at[pl.ds(e0, BLK)]], rs, sem)
        pltpu.async_copy(feats.at[dst_v.at[pl.ds(e0, BLK)]], rd, sem)

    def drain(rs, rd, sem):
        pltpu.make_async_copy(feats.at[src_v.at[pl.ds(0, BLK)]], rs, sem).wait()
        pltpu.make_async_copy(feats.at[dst_v.at[pl.ds(0, BLK)]], rd, sem).wait()

    def compute(b, rs, rd):
        def group(g, carry):
            le = g * 16
            for j in range(16):
                e = le + j
                acc0 = rs[e, pl.ds(0, 16)] * rd[e, pl.ds(0, 16)]
                acc1 = rs[e, pl.ds(16, 16)] * rd[e, pl.ds(16, 16)]
                for c in range(2, 8, 2):
                    acc0 = acc0 + rs[e, pl.ds(c * 16, 16)] * rd[e, pl.ds(c * 16, 16)]
                    acc1 = acc1 + rs[e, pl.ds((c + 1) * 16, 16)] * rd[e, pl.ds((c + 1) * 16, 16)]
                plsc.store_scatter(tr_v, [lane * 17 + j], acc0 + acc1)
            parts = [tr_v[pl.ds(l * 17, 16)] for l in range(16)]
            while len(parts) > 1:
                parts = [parts[i] + parts[i + 1] for i in range(0, len(parts), 2)]
            out_v[pl.ds(b * BLK + g * 16, 16)] = 1.0 / (1.0 + jnp.exp(-parts[0]))
            return carry

        lax.fori_loop(0, GRP, group, 0)

    start(0, rs_a, rd_a, sem_a)

    def pair(i, carry):
        b_even = 2 * i
        start(b_even + 1, rs_b, rd_b, sem_b)
        drain(rs_a, rd_a, sem_a)
        compute(b_even, rs_a, rd_a)
        start(b_even + 2, rs_a, rd_a, sem_a)
        drain(rs_b, rd_b, sem_b)
        compute(b_even + 1, rs_b, rd_b)
        return carry

    lax.fori_loop(0, (NBLK - 1) // 2, pair, 0)
    drain(rs_a, rd_a, sem_a)
    compute(NBLK - 1, rs_a, rd_a)
    pltpu.sync_copy(out_v, out.at[pl.ds(base, EPW)])


def kernel(node_feats, node_xy, node_adj_ids, edge_ids, W_node, b_node,
           W_edge, b_edge):
    feats, ns = _dense(node_feats, W_node, b_node.reshape(1, 1),
                       W_edge, b_edge.reshape(1, D_FEAT))
    node_scores = ns[:, 0]
    src = edge_ids[0].astype(jnp.int32)
    dst = edge_ids[1].astype(jnp.int32)
    edge_scores = _edge_kernel(feats, src, dst)
    return (node_scores, edge_scores)


# speedup vs baseline: 7.9323x; 1.0899x over previous
import functools

import jax
import jax.numpy as jnp
from jax import lax
from jax.experimental import pallas as pl
from jax.experimental.pallas import tpu as pltpu
from jax.experimental.pallas import tpu_sc as plsc

N_NODES = 10000
N_EDGES = 320000
D_FEAT = 128

NC = 2
NS = 16
NW = NC * NS
EPW = N_EDGES // NW
BLK = 80
NBLK = EPW // BLK
GRP = BLK // 16


def _dense_body(x_ref, wn_ref, bn_ref, we_ref, be_ref, feats_ref, ns_ref):
    x = x_ref[...]
    feats = jnp.dot(x, we_ref[...], preferred_element_type=jnp.float32)
    feats_ref[...] = (feats + be_ref[...]).astype(jnp.bfloat16)
    ns = jnp.dot(x, wn_ref[...], preferred_element_type=jnp.float32)
    ns_ref[...] = jax.nn.sigmoid(ns + bn_ref[...])


def _dense(node_feats, W_node, b_node, W_edge, b_edge):
    return pl.pallas_call(
        _dense_body,
        out_shape=(
            jax.ShapeDtypeStruct((N_NODES, D_FEAT), jnp.bfloat16),
            jax.ShapeDtypeStruct((N_NODES, 1), jnp.float32),
        ),
    )(node_feats, W_node, b_node, W_edge, b_edge)


_mesh = plsc.VectorSubcoreMesh(core_axis_name="c", subcore_axis_name="s")


@functools.partial(
    pl.kernel,
    mesh=_mesh,
    out_type=jax.ShapeDtypeStruct((N_EDGES,), jnp.float32),
    compiler_params=pltpu.CompilerParams(needs_layout_passes=False,
                                         use_tc_tiling_on_sc=False),
    scratch_types=[
        pltpu.VMEM((EPW,), jnp.int32),
        pltpu.VMEM((EPW,), jnp.int32),
        pltpu.VMEM((BLK, D_FEAT // 2), jnp.int32),
        pltpu.VMEM((BLK, D_FEAT // 2), jnp.int32),
        pltpu.VMEM((BLK, D_FEAT // 2), jnp.int32),
        pltpu.VMEM((BLK, D_FEAT // 2), jnp.int32),
        pltpu.VMEM((EPW,), jnp.float32),
        pltpu.VMEM((16 * 17,), jnp.float32),
        pltpu.SemaphoreType.DMA,
        pltpu.SemaphoreType.DMA,
    ],
)
def _edge_kernel(feats, src, dst, out, src_v, dst_v,
                 rs_a, rd_a, rs_b, rd_b, out_v, tr_v, sem_a, sem_b):
    wid = lax.axis_index("s") * NC + lax.axis_index("c")
    base = wid * EPW
    pltpu.sync_copy(src.at[pl.ds(base, EPW)], src_v)
    pltpu.sync_copy(dst.at[pl.ds(base, EPW)], dst_v)

    lane = lax.iota(jnp.int32, 16)

    def start(b, rs, rd, sem):
        e0 = b * BLK
        pltpu.async_copy(feats.at[src_v.at[pl.ds(e0, BLK)]], rs, sem)
        pltpu.async_copy(feats.at[dst_v.at[pl.ds(e0, BLK)]], rd, sem)

    def drain(rs, rd, sem):
        pltpu.make_async_copy(feats.at[src_v.at[pl.ds(0, BLK)]], rs, sem).wait()
        pltpu.make_async_copy(feats.at[dst_v.at[pl.ds(0, BLK)]], rd, sem).wait()

    def compute(b, rs, rd):
        def group(g, carry):
            le = g * 16
            for j in range(16):
                e = le + j
                acc0 = jnp.zeros((16,), jnp.float32)
                acc1 = jnp.zeros((16,), jnp.float32)
                for c in range(4):
                    pa = plsc.bitcast(rs[e, pl.ds(c * 16, 16)], jnp.bfloat16)
                    pb = plsc.bitcast(rd[e, pl.ds(c * 16, 16)], jnp.bfloat16)
                    u0, u1 = plsc.unpack(pa * pb,
                                         format=plsc.PackFormat.INTERLEAVED)
                    acc0 = acc0 + u0
                    acc1 = acc1 + u1
                plsc.store_scatter(tr_v, [lane * 17 + j], acc0 + acc1)
            parts = [tr_v[pl.ds(l * 17, 16)] for l in range(16)]
            while len(parts) > 1:
                parts = [parts[i] + parts[i + 1] for i in range(0, len(parts), 2)]
            out_v[pl.ds(b * BLK + g * 16, 16)] = 1.0 / (1.0 + jnp.exp(-parts[0]))
            return carry

        lax.fori_loop(0, GRP, group, 0)

    start(0, rs_a, rd_a, sem_a)

    def pair(i, carry):
        b_even = 2 * i
        start(b_even + 1, rs_b, rd_b, sem_b)
        drain(rs_a, rd_a, sem_a)
        compute(b_even, rs_a, rd_a)
        start(b_even + 2, rs_a, rd_a, sem_a)
        drain(rs_b, rd_b, sem_b)
        compute(b_even + 1, rs_b, rd_b)
        return carry

    lax.fori_loop(0, (NBLK - 1) // 2, pair, 0)
    drain(rs_a, rd_a, sem_a)
    compute(NBLK - 1, rs_a, rd_a)
    pltpu.sync_copy(out_v, out.at[pl.ds(base, EPW)])


def kernel(node_feats, node_xy, node_adj_ids, edge_ids, W_node, b_node,
           W_edge, b_edge):
    feats, ns = _dense(node_feats, W_node, b_node.reshape(1, 1),
                       W_edge, b_edge.reshape(1, D_FEAT))
    node_scores = ns[:, 0]
    eids = edge_ids.astype(jnp.int32)
    packed = jax.lax.bitcast_convert_type(
        feats.reshape(N_NODES, D_FEAT // 2, 2), jnp.int32)
    edge_scores = _edge_kernel(packed, eids[0], eids[1])
    return (node_scores, edge_scores)


# speedup vs baseline: 8.9901x; 1.1334x over previous
import functools

import jax
import jax.numpy as jnp
from jax import lax
from jax.experimental import pallas as pl
from jax.experimental.pallas import tpu as pltpu
from jax.experimental.pallas import tpu_sc as plsc

N_NODES = 10000
N_EDGES = 320000
D_FEAT = 128

NC = 2
NS = 16
NW = NC * NS
EPW = N_EDGES // NW
BLK = 80
NBLK = EPW // BLK
GRP = BLK // 16


def _dense_body(x_ref, wn_ref, bn_ref, we_ref, be_ref, packed_ref, ns_ref):
    x = x_ref[...]
    feats = jnp.dot(x, we_ref[...], preferred_element_type=jnp.float32)
    feats = feats + be_ref[...]
    lo = jax.lax.bitcast_convert_type(
        feats[:, : D_FEAT // 2].astype(jnp.bfloat16), jnp.uint16)
    hi = jax.lax.bitcast_convert_type(
        feats[:, D_FEAT // 2 :].astype(jnp.bfloat16), jnp.uint16)
    packed = (hi.astype(jnp.uint32) << 16) | lo.astype(jnp.uint32)
    packed_ref[...] = jax.lax.bitcast_convert_type(packed, jnp.int32)
    ns = jnp.dot(x, wn_ref[...], preferred_element_type=jnp.float32)
    ns_ref[...] = jax.nn.sigmoid(ns + bn_ref[...])


def _dense(node_feats, W_node, b_node, W_edge, b_edge):
    return pl.pallas_call(
        _dense_body,
        out_shape=(
            jax.ShapeDtypeStruct((N_NODES, D_FEAT // 2), jnp.int32),
            jax.ShapeDtypeStruct((N_NODES, 1), jnp.float32),
        ),
    )(node_feats, W_node, b_node, W_edge, b_edge)


_mesh = plsc.VectorSubcoreMesh(core_axis_name="c", subcore_axis_name="s")


@functools.partial(
    pl.kernel,
    mesh=_mesh,
    out_type=jax.ShapeDtypeStruct((N_EDGES,), jnp.float32),
    compiler_params=pltpu.CompilerParams(needs_layout_passes=False,
                                         use_tc_tiling_on_sc=False),
    scratch_types=[
        pltpu.VMEM((EPW,), jnp.int32),
        pltpu.VMEM((EPW,), jnp.int32),
        pltpu.VMEM((BLK, D_FEAT // 2), jnp.int32),
        pltpu.VMEM((BLK, D_FEAT // 2), jnp.int32),
        pltpu.VMEM((BLK, D_FEAT // 2), jnp.int32),
        pltpu.VMEM((BLK, D_FEAT // 2), jnp.int32),
        pltpu.VMEM((EPW,), jnp.float32),
        pltpu.VMEM((16 * 17,), jnp.float32),
        pltpu.SemaphoreType.DMA,
        pltpu.SemaphoreType.DMA,
    ],
)
def _edge_kernel(feats, src, dst, out, src_v, dst_v,
                 rs_a, rd_a, rs_b, rd_b, out_v, tr_v, sem_a, sem_b):
    wid = lax.axis_index("s") * NC + lax.axis_index("c")
    base = wid * EPW
    pltpu.sync_copy(src.at[pl.ds(base, EPW)], src_v)
    pltpu.sync_copy(dst.at[pl.ds(base, EPW)], dst_v)

    lane = lax.iota(jnp.int32, 16)

    def start(b, rs, rd, sem):
        e0 = b * BLK
        pltpu.async_copy(feats.at[src_v.at[pl.ds(e0, BLK)]], rs, sem)
        pltpu.async_copy(feats.at[dst_v.at[pl.ds(e0, BLK)]], rd, sem)

    def drain(rs, rd, sem):
        pltpu.make_async_copy(feats.at[src_v.at[pl.ds(0, BLK)]], rs, sem).wait()
        pltpu.make_async_copy(feats.at[dst_v.at[pl.ds(0, BLK)]], rd, sem).wait()

    def compute(b, rs, rd):
        def group(g, carry):
            le = g * 16
            for j in range(16):
                e = le + j
                acc0 = jnp.zeros((16,), jnp.float32)
                acc1 = jnp.zeros((16,), jnp.float32)
                for c in range(4):
                    pa = plsc.bitcast(rs[e, pl.ds(c * 16, 16)], jnp.bfloat16)
                    pb = plsc.bitcast(rd[e, pl.ds(c * 16, 16)], jnp.bfloat16)
                    u0, u1 = plsc.unpack(pa * pb,
                                         format=plsc.PackFormat.INTERLEAVED)
                    acc0 = acc0 + u0
                    acc1 = acc1 + u1
                plsc.store_scatter(tr_v, [lane * 17 + j], acc0 + acc1)
            parts = [tr_v[pl.ds(l * 17, 16)] for l in range(16)]
            while len(parts) > 1:
                parts = [parts[i] + parts[i + 1] for i in range(0, len(parts), 2)]
            out_v[pl.ds(b * BLK + g * 16, 16)] = 1.0 / (1.0 + jnp.exp(-parts[0]))
            return carry

        lax.fori_loop(0, GRP, group, 0)

    start(0, rs_a, rd_a, sem_a)

    def pair(i, carry):
        b_even = 2 * i
        start(b_even + 1, rs_b, rd_b, sem_b)
        drain(rs_a, rd_a, sem_a)
        compute(b_even, rs_a, rd_a)
        start(b_even + 2, rs_a, rd_a, sem_a)
        drain(rs_b, rd_b, sem_b)
        compute(b_even + 1, rs_b, rd_b)
        return carry

    lax.fori_loop(0, (NBLK - 1) // 2, pair, 0)
    drain(rs_a, rd_a, sem_a)
    compute(NBLK - 1, rs_a, rd_a)
    pltpu.sync_copy(out_v, out.at[pl.ds(base, EPW)])


def kernel(node_feats, node_xy, node_adj_ids, edge_ids, W_node, b_node,
           W_edge, b_edge):
    packed, ns = _dense(node_feats, W_node, b_node.reshape(1, 1),
                        W_edge, b_edge.reshape(1, D_FEAT))
    node_scores = ns[:, 0]
    eids = edge_ids.astype(jnp.int32)
    edge_scores = _edge_kernel(packed, eids[0], eids[1])
    return (node_scores, edge_scores)
